# baseline (speedup 1.0000x reference)
import functools

import jax
import jax.numpy as jnp
from jax import lax
from jax.experimental import pallas as pl
from jax.experimental.pallas import tpu as pltpu
from jax.experimental.pallas import tpu_sc as plsc

N = 10000
R = 8
H = 128
NL = 16
E = 320000

NC = 2
NS = 16
NW = NC * NS
ROWS_W = 80
EPW = ROWS_W * 128
EP = NW * EPW
EROWS = EP // 128
NSEG = N * R
NSEG_P = 80128
SEG_T = NSEG_P // NS
NP = 10112
NROW_T = NP // NS

BC = 64
NBC = EP // BC
C_B0 = 192
C_B1 = 128

BD = 128
NBD = EP // BD
D_B0 = 96
D_B1 = 64

_i32 = jnp.int32
_f32 = jnp.float32


def _mesh():
  return plsc.VectorSubcoreMesh(
      core_axis_name="c", subcore_axis_name="s", num_cores=NC, num_subcores=NS)


def _full16(v):
  return jnp.full((16,), v, dtype=_i32)


def _bcast_lane(vec, l):
  return lax.gather(
      vec, _full16(l).reshape(16, 1),
      lax.GatherDimensionNumbers(
          offset_dims=(), collapsed_slice_dims=(0,), start_index_map=(0,)),
      (1,), mode=lax.GatherScatterMode.PROMISE_IN_BOUNDS)


def _sc_counts(srcp, dstp, relp):
  @functools.partial(
      pl.kernel,
      out_type=(
          jax.ShapeDtypeStruct((NC * NSEG_P,), _f32),
          jax.ShapeDtypeStruct((EROWS, 128), _i32),
          jax.ShapeDtypeStruct((EROWS, 128), _i32),
          jax.ShapeDtypeStruct((EROWS, 128), _i32),
      ),
      mesh=_mesh(),
      scratch_types=(
          pltpu.VMEM((ROWS_W, 128), _i32),
          pltpu.VMEM((ROWS_W, 128), _i32),
          pltpu.VMEM((ROWS_W, 128), _i32),
          pltpu.VMEM((ROWS_W, 128), _i32),
          pltpu.VMEM((ROWS_W, 128), _i32),
          pltpu.VMEM((ROWS_W, 128), _i32),
          pltpu.VMEM((128,), _f32),
          pltpu.VMEM((SEG_T,), _f32),
          pltpu.VMEM_SHARED((NSEG_P,), _f32),
          pltpu.SemaphoreType.DMA,
      ),
  )
  def k(src_h, dst_h, rel_h, cnt_h, seg_h, gidx_h, gidx2_h,
        sb, db, rb, segb, gb, g2b, ones, zbuf, cnt_sh, sem):
    c = lax.axis_index("c")
    s = lax.axis_index("s")
    wid = s * NC + c
    wb = wid * ROWS_W

    zeros16 = jnp.zeros((16,), _f32)

    @pl.loop(0, SEG_T // 16)
    def _(i):
      zbuf[pl.ds(i * 16, 16)] = zeros16

    pltpu.sync_copy(zbuf, cnt_sh.at[pl.ds(s * SEG_T, SEG_T)])

    for k8 in range(8):
      ones[pl.ds(k8 * 16, 16)] = jnp.ones((16,), _f32)

    pltpu.sync_copy(src_h.at[pl.ds(wb, ROWS_W)], sb)
    pltpu.sync_copy(dst_h.at[pl.ds(wb, ROWS_W)], db)
    pltpu.sync_copy(rel_h.at[pl.ds(wb, ROWS_W)], rb)

    @pl.loop(0, ROWS_W)
    def _(j):
      for k8 in range(8):
        sl = pl.ds(k8 * 16, 16)
        sv = sb[j, sl]
        dv = db[j, sl]
        rv = rb[j, sl]
        segb[j, sl] = dv * R + rv
        gb[j, sl] = rv * N + sv
        g2b[j, sl] = sv * R + rv

    pltpu.sync_copy(segb, seg_h.at[pl.ds(wb, ROWS_W)])
    pltpu.sync_copy(gb, gidx_h.at[pl.ds(wb, ROWS_W)])
    pltpu.sync_copy(g2b, gidx2_h.at[pl.ds(wb, ROWS_W)])

    plsc.subcore_barrier()

    descs = [
        pltpu.async_copy(ones, cnt_sh.at[segb.at[j]], sem, add=True)
        for j in range(ROWS_W)
    ]
    for d in descs:
      d.wait()

    plsc.subcore_barrier()
    pltpu.sync_copy(cnt_sh.at[pl.ds(s * SEG_T, SEG_T)], zbuf)
    pltpu.sync_copy(zbuf, cnt_h.at[pl.ds(c * NSEG_P + s * SEG_T, SEG_T)])

  return k(srcp, dstp, relp)


def _ring_body(gidx_h, dst_h, seg_h, tab_h, inv_h, acc_sh,
               gring, dring, segring, sring, rbs, isems, gsems, ssems,
               nb, wb, compute_block):
  def istart(j, slot):
    pltpu.async_copy(gidx_h.at[pl.ds(wb + j, 1)], gring.at[pl.ds(slot, 1)],
                     isems[slot])
    pltpu.async_copy(dst_h.at[pl.ds(wb + j, 1)], dring.at[pl.ds(slot, 1)],
                     isems[slot])
    pltpu.async_copy(seg_h.at[pl.ds(wb + j, 1)], segring.at[pl.ds(slot, 1)],
                     isems[slot])

  def iwait(slot):
    for _ in range(3):
      pltpu.make_async_copy(gidx_h.at[pl.ds(wb, 1)],
                            gring.at[pl.ds(slot, 1)], isems[slot]).wait()

  def gstart(slot, rslot):
    pltpu.async_copy(tab_h.at[gring.at[slot]], rbs[rslot], gsems[rslot])
    pltpu.async_copy(inv_h.at[segring.at[slot]], sring.at[slot],
                     gsems[rslot])

  def gwait(rslot):
    pltpu.make_async_copy(tab_h.at[gring.at[0]], rbs[rslot],
                          gsems[rslot]).wait()
    pltpu.make_async_copy(inv_h.at[segring.at[0]], sring.at[0],
                          gsems[rslot]).wait()

  def sstart(slot, rslot):
    pltpu.async_copy(rbs[rslot], acc_sh.at[dring.at[slot]], ssems[rslot],
                     add=True)

  def swait(rslot):
    pltpu.make_async_copy(rbs[rslot], acc_sh.at[dring.at[0]],
                          ssems[rslot]).wait()

  for p in range(6):
    istart(p, p)
  iwait(0)
  gstart(0, 0)
  iwait(1)
  gstart(1, 1)

  plsc.subcore_barrier()

  @pl.loop(0, nb // 8)
  def _(q):
    for r in range(8):
      j = q * 8 + r
      rs = r % 4

      @pl.when(j >= 2)
      def _():
        swait((rs + 2) % 4)

      @pl.when(j + 2 < nb)
      def _():
        iwait((r + 2) % 8)
        gstart((r + 2) % 8, (rs + 2) % 4)

      @pl.when(j + 6 < nb)
      def _():
        istart(j + 6, (r + 6) % 8)

      gwait(rs)
      compute_block(r, rbs[rs])
      sstart(r, rs)

  swait(2)
  swait(3)


def _sc_layer1(gidx64, dstp64, seg64, w1f, inv):
  @functools.partial(
      pl.kernel,
      out_type=jax.ShapeDtypeStruct((NC * NP, H), _f32),
      mesh=_mesh(),
      scratch_types=(
          pltpu.VMEM((8, BC), _i32),
          pltpu.VMEM((8, BC), _i32),
          pltpu.VMEM((8, BC), _i32),
          pltpu.VMEM((8, BC), _f32),
          pltpu.VMEM((BC, H), _f32),
          pltpu.VMEM((BC, H), _f32),
          pltpu.VMEM((BC, H), _f32),
          pltpu.VMEM((BC, H), _f32),
          pltpu.VMEM_SHARED((NP, H), _f32),
          (pltpu.SemaphoreType.DMA,) * 8,
          (pltpu.SemaphoreType.DMA,) * 4,
          (pltpu.SemaphoreType.DMA,) * 4,
      ),
  )
  def k(gidx_h, dst_h, seg_h, w1_h, inv_h, hout_h,
        gring, dring, segring, sring, rb0, rb1, rb2, rb3, h_sh,
        isems, gsems, ssems):
    c = lax.axis_index("c")
    s = lax.axis_index("s")
    nb = jnp.where(c == 0, C_B0, C_B1)
    wb = s * (C_B0 + C_B1) + c * C_B0
    rbs = (rb0, rb1, rb2, rb3)

    zeros16 = jnp.zeros((16,), _f32)

    @pl.loop(0, BC)
    def _(i):
      for k8 in range(8):
        rb0[i, pl.ds(k8 * 16, 16)] = zeros16

    base = s * NROW_T
    off = 0
    for sz in [BC] * 9 + [NROW_T - 9 * BC]:
      pltpu.sync_copy(rb0.at[pl.ds(0, sz)], h_sh.at[pl.ds(base + off, sz)])
      off += sz

    def compute_block(r, rb):
      @pl.loop(0, BC, step=16)
      def _(e0):
        s_vec = sring[r, pl.ds(e0, 16)]
        for l in range(16):
          e = e0 + l
          sbc = _bcast_lane(s_vec, l)
          for k8 in range(8):
            sl = pl.ds(k8 * 16, 16)
            rb[e, sl] = rb[e, sl] * sbc

    _ring_body(gidx_h, dst_h, seg_h, w1_h, inv_h, h_sh,
               gring, dring, segring, sring, rbs, isems, gsems, ssems,
               nb, wb, compute_block)

    plsc.subcore_barrier()
    off = 0
    for sz in [BC] * 9 + [NROW_T - 9 * BC]:
      pltpu.sync_copy(h_sh.at[pl.ds(base + off, sz)], rb0.at[pl.ds(0, sz)])
      pltpu.sync_copy(rb0.at[pl.ds(0, sz)],
                      hout_h.at[pl.ds(c * NP + base + off, sz)])
      off += sz

  return k(gidx64, dstp64, seg64, w1f, inv)


def _sc_layer2(gidx2, dstp, seg, y16, inv):
  @functools.partial(
      pl.kernel,
      out_type=jax.ShapeDtypeStruct((NC * NP, NL), _f32),
      mesh=_mesh(),
      compiler_params=pltpu.CompilerParams(use_tc_tiling_on_sc=False),
      scratch_types=(
          pltpu.VMEM((8, BD), _i32),
          pltpu.VMEM((8, BD), _i32),
          pltpu.VMEM((8, BD), _i32),
          pltpu.VMEM((8, BD), _f32),
          pltpu.VMEM((BD, NL), _f32),
          pltpu.VMEM((BD, NL), _f32),
          pltpu.VMEM((BD, NL), _f32),
          pltpu.VMEM((BD, NL), _f32),
          pltpu.VMEM_SHARED((NP, NL), _f32),
          (pltpu.SemaphoreType.DMA,) * 8,
          (pltpu.SemaphoreType.DMA,) * 4,
          (pltpu.SemaphoreType.DMA,) * 4,
      ),
  )
  def k(gidx_h, dst_h, seg_h, y_h, inv_h, oout_h,
        gring, dring, segring, sring, rb0, rb1, rb2, rb3, o_sh,
        isems, gsems, ssems):
    c = lax.axis_index("c")
    s = lax.axis_index("s")
    nb = jnp.where(c == 0, D_B0, D_B1)
    wb = s * (D_B0 + D_B1) + c * D_B0
    rbs = (rb0, rb1, rb2, rb3)

    zeros16 = jnp.zeros((16,), _f32)

    @pl.loop(0, BD)
    def _(i):
      rb0[i, pl.ds(0, 16)] = zeros16

    base = s * NROW_T
    for off, sz in ((0, BD), (BD, BD), (2 * BD, BD), (3 * BD, BD),
                    (4 * BD, NROW_T - 4 * BD)):
      pltpu.sync_copy(rb0.at[pl.ds(0, sz)], o_sh.at[pl.ds(base + off, sz)])

    def compute_block(r, rb):
      @pl.loop(0, BD, step=16)
      def _(e0):
        s_vec = sring[r, pl.ds(e0, 16)]
        for l in range(16):
          e = e0 + l
          sbc = _bcast_lane(s_vec, l)
          rb[e, pl.ds(0, 16)] = rb[e, pl.ds(0, 16)] * sbc

    _ring_body(gidx_h, dst_h, seg_h, y_h, inv_h, o_sh,
               gring, dring, segring, sring, rbs, isems, gsems, ssems,
               nb, wb, compute_block)

    plsc.subcore_barrier()
    for off, sz in ((0, BD), (BD, BD), (2 * BD, BD), (3 * BD, BD),
                    (4 * BD, NROW_T - 4 * BD)):
      pltpu.sync_copy(o_sh.at[pl.ds(base + off, sz)], rb0.at[pl.ds(0, sz)])
      pltpu.sync_copy(rb0.at[pl.ds(0, sz)],
                      oout_h.at[pl.ds(c * NP + base + off, sz)])

  return k(gidx2, dstp, seg, y16, inv)


def _tc_inv(cnt):
  def body(cnt_ref, inv_ref):
    tot = cnt_ref[0] + cnt_ref[1]
    inv_ref[...] = 1.0 / jnp.maximum(tot, 1.0)

  return pl.pallas_call(
      body,
      out_shape=jax.ShapeDtypeStruct((NSEG_P // 128, 128), _f32),
  )(cnt.reshape(NC, NSEG_P // 128, 128))


def _tc_dense(hpart, root1, bias1, w2cat, root2):
  def body(h_ref, r1_ref, b1_ref, w2_ref, rt2_ref, y_ref, xr_ref):
    x = h_ref[:N, :] + h_ref[NP:NP + N, :] + r1_ref[...] + b1_ref[...]
    x = jnp.maximum(x, 0.0)
    y_ref[...] = jnp.dot(x, w2_ref[...], preferred_element_type=_f32)
    xr_ref[...] = jnp.dot(x, rt2_ref[...], preferred_element_type=_f32)

  return pl.pallas_call(
      body,
      out_shape=(
          jax.ShapeDtypeStruct((N, R * NL), _f32),
          jax.ShapeDtypeStruct((N, NL), _f32),
      ),
  )(hpart, root1, bias1.reshape(1, H), w2cat, root2)


def _tc_final(opart, xr, bias2):
  def body(o_ref, xr_ref, b2_ref, out_ref):
    t = o_ref[:N, :] + o_ref[NP:NP + N, :] + xr_ref[...] + b2_ref[...]
    out_ref[...] = jax.nn.sigmoid(t)

  return pl.pallas_call(
      body,
      out_shape=jax.ShapeDtypeStruct((N, NL), _f32),
  )(opart, xr, bias2.reshape(1, NL))


def kernel(edge_index, edge_type, weight1, root1, bias1, weight2, root2, bias2):
  src = edge_index[0].astype(_i32)
  dst = edge_index[1].astype(_i32)
  rel = edge_type.astype(_i32)

  pad = EP - E
  srcp = jnp.concatenate([src, jnp.zeros((pad,), _i32)]).reshape(EROWS, 128)
  dstp = jnp.concatenate([dst, jnp.full((pad,), N, _i32)]).reshape(EROWS, 128)
  relp = jnp.concatenate([rel, jnp.zeros((pad,), _i32)]).reshape(EROWS, 128)

  w1f = weight1.reshape(NSEG, H)
  w2cat = weight2.transpose(1, 0, 2).reshape(H, R * NL)

  cnt, seg, gidx, gidx2 = _sc_counts(srcp, dstp, relp)
  inv = _tc_inv(cnt).reshape(NSEG_P)
  hpart = _sc_layer1(gidx.reshape(EP // BC, BC), dstp.reshape(EP // BC, BC),
                     seg.reshape(EP // BC, BC), w1f, inv)
  y2d, xr = _tc_dense(hpart, root1, bias1, w2cat, root2)
  y16 = y2d.reshape(NSEG, NL)
  opart = _sc_layer2(gidx2, dstp, seg, y16, inv)
  return _tc_final(opart, xr, bias2)


# speedup vs baseline: 1.0249x; 1.0249x over previous
import functools

import jax
import jax.numpy as jnp
from jax import lax
from jax.experimental import pallas as pl
from jax.experimental.pallas import tpu as pltpu
from jax.experimental.pallas import tpu_sc as plsc

N = 10000
R = 8
H = 128
NL = 16
E = 320000

NC = 2
NS = 16
NW = NC * NS
ROWS_W = 80
EPW = ROWS_W * 128
EP = NW * EPW
EROWS = EP // 128
NSEG = N * R
NSEG_P = 80128
SEG_T = NSEG_P // NS
NP = 10112
NROW_T = NP // NS

BC = 64
NBC = EP // BC
C_B0 = 272
C_B1 = 48

BD = 128
NBD = EP // BD
D_B0 = 96
D_B1 = 64

_i32 = jnp.int32
_f32 = jnp.float32


def _mesh():
  return plsc.VectorSubcoreMesh(
      core_axis_name="c", subcore_axis_name="s", num_cores=NC, num_subcores=NS)


def _full16(v):
  return jnp.full((16,), v, dtype=_i32)


def _bcast_lane(vec, l):
  return lax.gather(
      vec, _full16(l).reshape(16, 1),
      lax.GatherDimensionNumbers(
          offset_dims=(), collapsed_slice_dims=(0,), start_index_map=(0,)),
      (1,), mode=lax.GatherScatterMode.PROMISE_IN_BOUNDS)


def _sc_counts(srcp, dstp, relp):
  @functools.partial(
      pl.kernel,
      out_type=(
          jax.ShapeDtypeStruct((NC * NSEG_P,), _f32),
          jax.ShapeDtypeStruct((EROWS, 128), _i32),
          jax.ShapeDtypeStruct((EROWS, 128), _i32),
          jax.ShapeDtypeStruct((EROWS, 128), _i32),
      ),
      mesh=_mesh(),
      scratch_types=(
          pltpu.VMEM((ROWS_W, 128), _i32),
          pltpu.VMEM((ROWS_W, 128), _i32),
          pltpu.VMEM((ROWS_W, 128), _i32),
          pltpu.VMEM((ROWS_W, 128), _i32),
          pltpu.VMEM((ROWS_W, 128), _i32),
          pltpu.VMEM((ROWS_W, 128), _i32),
          pltpu.VMEM((128,), _f32),
          pltpu.VMEM((SEG_T,), _f32),
          pltpu.VMEM_SHARED((NSEG_P,), _f32),
          pltpu.SemaphoreType.DMA,
      ),
  )
  def k(src_h, dst_h, rel_h, cnt_h, seg_h, gidx_h, gidx2_h,
        sb, db, rb, segb, gb, g2b, ones, zbuf, cnt_sh, sem):
    c = lax.axis_index("c")
    s = lax.axis_index("s")
    wid = s * NC + c
    wb = wid * ROWS_W

    zeros16 = jnp.zeros((16,), _f32)

    @pl.loop(0, SEG_T // 16)
    def _(i):
      zbuf[pl.ds(i * 16, 16)] = zeros16

    pltpu.sync_copy(zbuf, cnt_sh.at[pl.ds(s * SEG_T, SEG_T)])

    for k8 in range(8):
      ones[pl.ds(k8 * 16, 16)] = jnp.ones((16,), _f32)

    pltpu.sync_copy(src_h.at[pl.ds(wb, ROWS_W)], sb)
    pltpu.sync_copy(dst_h.at[pl.ds(wb, ROWS_W)], db)
    pltpu.sync_copy(rel_h.at[pl.ds(wb, ROWS_W)], rb)

    @pl.loop(0, ROWS_W)
    def _(j):
      for k8 in range(8):
        sl = pl.ds(k8 * 16, 16)
        sv = sb[j, sl]
        dv = db[j, sl]
        rv = rb[j, sl]
        segb[j, sl] = dv * R + rv
        gb[j, sl] = rv * N + sv
        g2b[j, sl] = sv * R + rv

    pltpu.sync_copy(segb, seg_h.at[pl.ds(wb, ROWS_W)])
    pltpu.sync_copy(gb, gidx_h.at[pl.ds(wb, ROWS_W)])
    pltpu.sync_copy(g2b, gidx2_h.at[pl.ds(wb, ROWS_W)])

    plsc.subcore_barrier()

    descs = [
        pltpu.async_copy(ones, cnt_sh.at[segb.at[j]], sem, add=True)
        for j in range(ROWS_W)
    ]
    for d in descs:
      d.wait()

    plsc.subcore_barrier()
    pltpu.sync_copy(cnt_sh.at[pl.ds(s * SEG_T, SEG_T)], zbuf)
    pltpu.sync_copy(zbuf, cnt_h.at[pl.ds(c * NSEG_P + s * SEG_T, SEG_T)])

  return k(srcp, dstp, relp)


def _ring_body(gidx_h, dst_h, seg_h, tab_h, inv_h, acc_sh,
               gring, dring, segring, sring, rbs, isems, gsems, ssems,
               nb, wb, compute_block):
  def istart(j, slot):
    pltpu.async_copy(gidx_h.at[pl.ds(wb + j, 1)], gring.at[pl.ds(slot, 1)],
                     isems[slot])
    pltpu.async_copy(dst_h.at[pl.ds(wb + j, 1)], dring.at[pl.ds(slot, 1)],
                     isems[slot])
    pltpu.async_copy(seg_h.at[pl.ds(wb + j, 1)], segring.at[pl.ds(slot, 1)],
                     isems[slot])

  def iwait(slot):
    for _ in range(3):
      pltpu.make_async_copy(gidx_h.at[pl.ds(wb, 1)],
                            gring.at[pl.ds(slot, 1)], isems[slot]).wait()

  def gstart(slot, rslot):
    pltpu.async_copy(tab_h.at[gring.at[slot]], rbs[rslot], gsems[rslot])
    pltpu.async_copy(inv_h.at[segring.at[slot]], sring.at[slot],
                     gsems[rslot])

  def gwait(rslot):
    pltpu.make_async_copy(tab_h.at[gring.at[0]], rbs[rslot],
                          gsems[rslot]).wait()
    pltpu.make_async_copy(inv_h.at[segring.at[0]], sring.at[0],
                          gsems[rslot]).wait()

  def sstart(slot, rslot):
    pltpu.async_copy(rbs[rslot], acc_sh.at[dring.at[slot]], ssems[rslot],
                     add=True)

  def swait(rslot):
    pltpu.make_async_copy(rbs[rslot], acc_sh.at[dring.at[0]],
                          ssems[rslot]).wait()

  for p in range(6):
    istart(p, p)
  iwait(0)
  gstart(0, 0)
  iwait(1)
  gstart(1, 1)

  plsc.subcore_barrier()

  @pl.loop(0, nb // 8)
  def _(q):
    for r in range(8):
      j = q * 8 + r
      rs = r % 4

      @pl.when(j >= 2)
      def _():
        swait((rs + 2) % 4)

      @pl.when(j + 2 < nb)
      def _():
        iwait((r + 2) % 8)
        gstart((r + 2) % 8, (rs + 2) % 4)

      @pl.when(j + 6 < nb)
      def _():
        istart(j + 6, (r + 6) % 8)

      gwait(rs)
      compute_block(r, rbs[rs])
      sstart(r, rs)

  swait(2)
  swait(3)


def _sc_layer1(gidx64, dstp64, seg64, w1f, inv):
  @functools.partial(
      pl.kernel,
      out_type=jax.ShapeDtypeStruct((NC * NP, H), _f32),
      mesh=_mesh(),
      scratch_types=(
          pltpu.VMEM((8, BC), _i32),
          pltpu.VMEM((8, BC), _i32),
          pltpu.VMEM((8, BC), _i32),
          pltpu.VMEM((8, BC), _f32),
          pltpu.VMEM((BC, H), _f32),
          pltpu.VMEM((BC, H), _f32),
          pltpu.VMEM((BC, H), _f32),
          pltpu.VMEM((BC, H), _f32),
          pltpu.VMEM_SHARED((NP, H), _f32),
          (pltpu.SemaphoreType.DMA,) * 8,
          (pltpu.SemaphoreType.DMA,) * 4,
          (pltpu.SemaphoreType.DMA,) * 4,
      ),
  )
  def k(gidx_h, dst_h, seg_h, w1_h, inv_h, hout_h,
        gring, dring, segring, sring, rb0, rb1, rb2, rb3, h_sh,
        isems, gsems, ssems):
    c = lax.axis_index("c")
    s = lax.axis_index("s")
    nb = jnp.where(c == 0, C_B0, C_B1)
    wb = s * (C_B0 + C_B1) + c * C_B0
    rbs = (rb0, rb1, rb2, rb3)

    zeros16 = jnp.zeros((16,), _f32)

    @pl.loop(0, BC)
    def _(i):
      for k8 in range(8):
        rb0[i, pl.ds(k8 * 16, 16)] = zeros16

    base = s * NROW_T
    off = 0
    for sz in [BC] * 9 + [NROW_T - 9 * BC]:
      pltpu.sync_copy(rb0.at[pl.ds(0, sz)], h_sh.at[pl.ds(base + off, sz)])
      off += sz

    def compute_block(r, rb):
      @pl.loop(0, BC, step=16)
      def _(e0):
        s_vec = sring[r, pl.ds(e0, 16)]
        for l in range(16):
          e = e0 + l
          sbc = _bcast_lane(s_vec, l)
          for k8 in range(8):
            sl = pl.ds(k8 * 16, 16)
            rb[e, sl] = rb[e, sl] * sbc

    _ring_body(gidx_h, dst_h, seg_h, w1_h, inv_h, h_sh,
               gring, dring, segring, sring, rbs, isems, gsems, ssems,
               nb, wb, compute_block)

    plsc.subcore_barrier()
    off = 0
    for sz in [BC] * 9 + [NROW_T - 9 * BC]:
      pltpu.sync_copy(h_sh.at[pl.ds(base + off, sz)], rb0.at[pl.ds(0, sz)])
      pltpu.sync_copy(rb0.at[pl.ds(0, sz)],
                      hout_h.at[pl.ds(c * NP + base + off, sz)])
      off += sz

  return k(gidx64, dstp64, seg64, w1f, inv)


def _sc_layer2(gidx2, dstp, seg, y16, inv):
  @functools.partial(
      pl.kernel,
      out_type=jax.ShapeDtypeStruct((NC * NP, NL), _f32),
      mesh=_mesh(),
      compiler_params=pltpu.CompilerParams(use_tc_tiling_on_sc=False),
      scratch_types=(
          pltpu.VMEM((8, BD), _i32),
          pltpu.VMEM((8, BD), _i32),
          pltpu.VMEM((8, BD), _i32),
          pltpu.VMEM((8, BD), _f32),
          pltpu.VMEM((BD, NL), _f32),
          pltpu.VMEM((BD, NL), _f32),
          pltpu.VMEM((BD, NL), _f32),
          pltpu.VMEM((BD, NL), _f32),
          pltpu.VMEM_SHARED((NP, NL), _f32),
          (pltpu.SemaphoreType.DMA,) * 8,
          (pltpu.SemaphoreType.DMA,) * 4,
          (pltpu.SemaphoreType.DMA,) * 4,
      ),
  )
  def k(gidx_h, dst_h, seg_h, y_h, inv_h, oout_h,
        gring, dring, segring, sring, rb0, rb1, rb2, rb3, o_sh,
        isems, gsems, ssems):
    c = lax.axis_index("c")
    s = lax.axis_index("s")
    nb = jnp.where(c == 0, D_B0, D_B1)
    wb = s * (D_B0 + D_B1) + c * D_B0
    rbs = (rb0, rb1, rb2, rb3)

    zeros16 = jnp.zeros((16,), _f32)

    @pl.loop(0, BD)
    def _(i):
      rb0[i, pl.ds(0, 16)] = zeros16

    base = s * NROW_T
    for off, sz in ((0, BD), (BD, BD), (2 * BD, BD), (3 * BD, BD),
                    (4 * BD, NROW_T - 4 * BD)):
      pltpu.sync_copy(rb0.at[pl.ds(0, sz)], o_sh.at[pl.ds(base + off, sz)])

    def compute_block(r, rb):
      @pl.loop(0, BD, step=16)
      def _(e0):
        s_vec = sring[r, pl.ds(e0, 16)]
        for l in range(16):
          e = e0 + l
          sbc = _bcast_lane(s_vec, l)
          rb[e, pl.ds(0, 16)] = rb[e, pl.ds(0, 16)] * sbc

    _ring_body(gidx_h, dst_h, seg_h, y_h, inv_h, o_sh,
               gring, dring, segring, sring, rbs, isems, gsems, ssems,
               nb, wb, compute_block)

    plsc.subcore_barrier()
    for off, sz in ((0, BD), (BD, BD), (2 * BD, BD), (3 * BD, BD),
                    (4 * BD, NROW_T - 4 * BD)):
      pltpu.sync_copy(o_sh.at[pl.ds(base + off, sz)], rb0.at[pl.ds(0, sz)])
      pltpu.sync_copy(rb0.at[pl.ds(0, sz)],
                      oout_h.at[pl.ds(c * NP + base + off, sz)])

  return k(gidx2, dstp, seg, y16, inv)


def _tc_inv(cnt):
  def body(cnt_ref, inv_ref):
    tot = cnt_ref[0] + cnt_ref[1]
    inv_ref[...] = 1.0 / jnp.maximum(tot, 1.0)

  return pl.pallas_call(
      body,
      out_shape=jax.ShapeDtypeStruct((NSEG_P // 128, 128), _f32),
  )(cnt.reshape(NC, NSEG_P // 128, 128))


def _tc_dense(hpart, root1, bias1, w2cat, root2):
  def body(h_ref, r1_ref, b1_ref, w2_ref, rt2_ref, y_ref, xr_ref):
    x = h_ref[:N, :] + h_ref[NP:NP + N, :] + r1_ref[...] + b1_ref[...]
    x = jnp.maximum(x, 0.0)
    y_ref[...] = jnp.dot(x, w2_ref[...], preferred_element_type=_f32)
    xr_ref[...] = jnp.dot(x, rt2_ref[...], preferred_element_type=_f32)

  return pl.pallas_call(
      body,
      out_shape=(
          jax.ShapeDtypeStruct((N, R * NL), _f32),
          jax.ShapeDtypeStruct((N, NL), _f32),
      ),
  )(hpart, root1, bias1.reshape(1, H), w2cat, root2)


def _tc_final(opart, xr, bias2):
  def body(o_ref, xr_ref, b2_ref, out_ref):
    t = o_ref[:N, :] + o_ref[NP:NP + N, :] + xr_ref[...] + b2_ref[...]
    out_ref[...] = jax.nn.sigmoid(t)

  return pl.pallas_call(
      body,
      out_shape=jax.ShapeDtypeStruct((N, NL), _f32),
  )(opart, xr, bias2.reshape(1, NL))


def kernel(edge_index, edge_type, weight1, root1, bias1, weight2, root2, bias2):
  src = edge_index[0].astype(_i32)
  dst = edge_index[1].astype(_i32)
  rel = edge_type.astype(_i32)

  pad = EP - E
  srcp = jnp.concatenate([src, jnp.zeros((pad,), _i32)]).reshape(EROWS, 128)
  dstp = jnp.concatenate([dst, jnp.full((pad,), N, _i32)]).reshape(EROWS, 128)
  relp = jnp.concatenate([rel, jnp.zeros((pad,), _i32)]).reshape(EROWS, 128)

  w1f = weight1.reshape(NSEG, H)
  w2cat = weight2.transpose(1, 0, 2).reshape(H, R * NL)

  cnt, seg, gidx, gidx2 = _sc_counts(srcp, dstp, relp)
  inv = _tc_inv(cnt).reshape(NSEG_P)
  hpart = _sc_layer1(gidx.reshape(EP // BC, BC), dstp.reshape(EP // BC, BC),
                     seg.reshape(EP // BC, BC), w1f, inv)
  y2d, xr = _tc_dense(hpart, root1, bias1, w2cat, root2)
  y16 = y2d.reshape(NSEG, NL)
  opart = _sc_layer2(gidx2, dstp, seg, y16, inv)
  return _tc_final(opart, xr, bias2)


# speedup vs baseline: 1.1759x; 1.1473x over previous
import functools

import jax
import jax.numpy as jnp
from jax import lax
from jax.experimental import pallas as pl
from jax.experimental.pallas import tpu as pltpu
from jax.experimental.pallas import tpu_sc as plsc

N = 10000
R = 8
H = 128
NL = 16
E = 320000

NC = 2
NS = 16
NW = NC * NS
ROWS_W = 80
EPW = ROWS_W * 128
EP = NW * EPW
EROWS = EP // 128
NSEG = N * R
NSEG_P = 80128
SEG_T = NSEG_P // NS
NP = 10112
NROW_T = NP // NS

BC = 64
NBC = EP // BC
C_B0 = 240
C_B1 = 80

BD = 128
NBD = EP // BD
D_B0 = 112
D_B1 = 48

_i32 = jnp.int32
_f32 = jnp.float32


def _mesh():
  return plsc.VectorSubcoreMesh(
      core_axis_name="c", subcore_axis_name="s", num_cores=NC, num_subcores=NS)


def _full16(v):
  return jnp.full((16,), v, dtype=_i32)


def _bcast_lane(vec, l):
  return lax.gather(
      vec, _full16(l).reshape(16, 1),
      lax.GatherDimensionNumbers(
          offset_dims=(), collapsed_slice_dims=(0,), start_index_map=(0,)),
      (1,), mode=lax.GatherScatterMode.PROMISE_IN_BOUNDS)


def _sc_counts(srcp, dstp, relp):
  @functools.partial(
      pl.kernel,
      out_type=(
          jax.ShapeDtypeStruct((NSEG_P,), _f32),
          jax.ShapeDtypeStruct((NSEG_P,), _f32),
          jax.ShapeDtypeStruct((EROWS, 128), _i32),
          jax.ShapeDtypeStruct((EROWS, 128), _i32),
          jax.ShapeDtypeStruct((EROWS, 128), _i32),
      ),
      mesh=_mesh(),
      scratch_types=(
          pltpu.VMEM((ROWS_W, 128), _i32),
          pltpu.VMEM((ROWS_W, 128), _i32),
          pltpu.VMEM((ROWS_W, 128), _i32),
          pltpu.VMEM((ROWS_W, 128), _i32),
          pltpu.VMEM((ROWS_W, 128), _i32),
          pltpu.VMEM((ROWS_W, 128), _i32),
          pltpu.VMEM((128,), _f32),
          pltpu.VMEM((SEG_T,), _f32),
          pltpu.VMEM_SHARED((NSEG_P,), _f32),
          pltpu.SemaphoreType.DMA,
      ),
  )
  def k(src_h, dst_h, rel_h, cnt0_h, cnt1_h, seg_h, gidx_h, gidx2_h,
        sb, db, rb, segb, gb, g2b, ones, zbuf, cnt_sh, sem):
    c = lax.axis_index("c")
    s = lax.axis_index("s")
    wid = s * NC + c
    wb = wid * ROWS_W

    zeros16 = jnp.zeros((16,), _f32)

    @pl.loop(0, SEG_T // 16)
    def _(i):
      zbuf[pl.ds(i * 16, 16)] = zeros16

    pltpu.sync_copy(zbuf, cnt_sh.at[pl.ds(s * SEG_T, SEG_T)])

    for k8 in range(8):
      ones[pl.ds(k8 * 16, 16)] = jnp.ones((16,), _f32)

    pltpu.sync_copy(src_h.at[pl.ds(wb, ROWS_W)], sb)
    pltpu.sync_copy(dst_h.at[pl.ds(wb, ROWS_W)], db)
    pltpu.sync_copy(rel_h.at[pl.ds(wb, ROWS_W)], rb)

    @pl.loop(0, ROWS_W)
    def _(j):
      for k8 in range(8):
        sl = pl.ds(k8 * 16, 16)
        sv = sb[j, sl]
        dv = db[j, sl]
        rv = rb[j, sl]
        segb[j, sl] = dv * R + rv
        gb[j, sl] = rv * N + sv
        g2b[j, sl] = sv * R + rv

    pltpu.sync_copy(segb, seg_h.at[pl.ds(wb, ROWS_W)])
    pltpu.sync_copy(gb, gidx_h.at[pl.ds(wb, ROWS_W)])
    pltpu.sync_copy(g2b, gidx2_h.at[pl.ds(wb, ROWS_W)])

    plsc.subcore_barrier()

    descs = [
        pltpu.async_copy(ones, cnt_sh.at[segb.at[j]], sem, add=True)
        for j in range(ROWS_W)
    ]
    for d in descs:
      d.wait()

    plsc.subcore_barrier()
    pltpu.sync_copy(cnt_sh.at[pl.ds(s * SEG_T, SEG_T)], zbuf)

    @pl.when(c == 0)
    def _():
      pltpu.sync_copy(zbuf, cnt0_h.at[pl.ds(s * SEG_T, SEG_T)])

    @pl.when(c == 1)
    def _():
      pltpu.sync_copy(zbuf, cnt1_h.at[pl.ds(s * SEG_T, SEG_T)])

  return k(srcp, dstp, relp)


def _ring_body(gidx_h, dst_h, seg_h, tab_h, cnt0_h, cnt1_h, acc_sh,
               gring, dring, segring, c0ring, c1ring, rbs,
               isems, gsems, ssems, nb, wb, compute_block):
  def istart(j, slot):
    pltpu.async_copy(gidx_h.at[pl.ds(wb + j, 1)], gring.at[pl.ds(slot, 1)],
                     isems[slot])
    pltpu.async_copy(dst_h.at[pl.ds(wb + j, 1)], dring.at[pl.ds(slot, 1)],
                     isems[slot])
    pltpu.async_copy(seg_h.at[pl.ds(wb + j, 1)], segring.at[pl.ds(slot, 1)],
                     isems[slot])

  def iwait(slot):
    for _ in range(3):
      pltpu.make_async_copy(gidx_h.at[pl.ds(wb, 1)],
                            gring.at[pl.ds(slot, 1)], isems[slot]).wait()

  def gstart(slot, rslot):
    pltpu.async_copy(tab_h.at[gring.at[slot]], rbs[rslot], gsems[rslot])
    pltpu.async_copy(cnt0_h.at[segring.at[slot]], c0ring.at[slot],
                     gsems[rslot])
    pltpu.async_copy(cnt1_h.at[segring.at[slot]], c1ring.at[slot],
                     gsems[rslot])

  def gwait(rslot):
    pltpu.make_async_copy(tab_h.at[gring.at[0]], rbs[rslot],
                          gsems[rslot]).wait()
    for _ in range(2):
      pltpu.make_async_copy(cnt0_h.at[segring.at[0]], c0ring.at[0],
                            gsems[rslot]).wait()

  def sstart(slot, rslot):
    pltpu.async_copy(rbs[rslot], acc_sh.at[dring.at[slot]], ssems[rslot],
                     add=True)

  def swait(rslot):
    pltpu.make_async_copy(rbs[rslot], acc_sh.at[dring.at[0]],
                          ssems[rslot]).wait()

  for p in range(6):
    istart(p, p)
  iwait(0)
  gstart(0, 0)
  iwait(1)
  gstart(1, 1)

  plsc.subcore_barrier()

  @pl.loop(0, nb // 8)
  def _(q):
    for r in range(8):
      j = q * 8 + r
      rs = r % 4

      @pl.when(j >= 2)
      def _():
        swait((rs + 2) % 4)

      @pl.when(j + 2 < nb)
      def _():
        iwait((r + 2) % 8)
        gstart((r + 2) % 8, (rs + 2) % 4)

      @pl.when(j + 6 < nb)
      def _():
        istart(j + 6, (r + 6) % 8)

      gwait(rs)
      compute_block(r, rbs[rs])
      sstart(r, rs)

  swait(2)
  swait(3)


def _sc_layer1(gidx64, dstp64, seg64, w1f, cnt0, cnt1):
  @functools.partial(
      pl.kernel,
      out_type=jax.ShapeDtypeStruct((NC * NP, H), _f32),
      mesh=_mesh(),
      scratch_types=(
          pltpu.VMEM((8, BC), _i32),
          pltpu.VMEM((8, BC), _i32),
          pltpu.VMEM((8, BC), _i32),
          pltpu.VMEM((8, BC), _f32),
          pltpu.VMEM((8, BC), _f32),
          pltpu.VMEM((BC, H), _f32),
          pltpu.VMEM((BC, H), _f32),
          pltpu.VMEM((BC, H), _f32),
          pltpu.VMEM((BC, H), _f32),
          pltpu.VMEM_SHARED((NP, H), _f32),
          (pltpu.SemaphoreType.DMA,) * 8,
          (pltpu.SemaphoreType.DMA,) * 4,
          (pltpu.SemaphoreType.DMA,) * 4,
      ),
  )
  def k(gidx_h, dst_h, seg_h, w1_h, cnt0_h, cnt1_h, hout_h,
        gring, dring, segring, c0ring, c1ring, rb0, rb1, rb2, rb3, h_sh,
        isems, gsems, ssems):
    c = lax.axis_index("c")
    s = lax.axis_index("s")
    nb = jnp.where(c == 0, C_B0, C_B1)
    wb = s * (C_B0 + C_B1) + c * C_B0
    rbs = (rb0, rb1, rb2, rb3)

    zeros16 = jnp.zeros((16,), _f32)

    @pl.loop(0, BC)
    def _(i):
      for k8 in range(8):
        rb0[i, pl.ds(k8 * 16, 16)] = zeros16

    base = s * NROW_T
    off = 0
    for sz in [BC] * 9 + [NROW_T - 9 * BC]:
      pltpu.sync_copy(rb0.at[pl.ds(0, sz)], h_sh.at[pl.ds(base + off, sz)])
      off += sz

    def compute_block(r, rb):
      @pl.loop(0, BC, step=16)
      def _(e0):
        ctot = c0ring[r, pl.ds(e0, 16)] + c1ring[r, pl.ds(e0, 16)]
        s_vec = 1.0 / jnp.maximum(ctot, 1.0)
        for l in range(16):
          e = e0 + l
          sbc = _bcast_lane(s_vec, l)
          for k8 in range(8):
            sl = pl.ds(k8 * 16, 16)
            rb[e, sl] = rb[e, sl] * sbc

    _ring_body(gidx_h, dst_h, seg_h, w1_h, cnt0_h, cnt1_h, h_sh,
               gring, dring, segring, c0ring, c1ring, rbs,
               isems, gsems, ssems, nb, wb, compute_block)

    plsc.subcore_barrier()
    off = 0
    for sz in [BC] * 9 + [NROW_T - 9 * BC]:
      pltpu.sync_copy(h_sh.at[pl.ds(base + off, sz)], rb0.at[pl.ds(0, sz)])
      pltpu.sync_copy(rb0.at[pl.ds(0, sz)],
                      hout_h.at[pl.ds(c * NP + base + off, sz)])
      off += sz

  return k(gidx64, dstp64, seg64, w1f, cnt0, cnt1)


def _sc_layer2(gidx2, dstp, seg, y16, cnt0, cnt1):
  @functools.partial(
      pl.kernel,
      out_type=jax.ShapeDtypeStruct((NC * NP, NL), _f32),
      mesh=_mesh(),
      compiler_params=pltpu.CompilerParams(use_tc_tiling_on_sc=False),
      scratch_types=(
          pltpu.VMEM((8, BD), _i32),
          pltpu.VMEM((8, BD), _i32),
          pltpu.VMEM((8, BD), _i32),
          pltpu.VMEM((8, BD), _f32),
          pltpu.VMEM((8, BD), _f32),
          pltpu.VMEM((BD, NL), _f32),
          pltpu.VMEM((BD, NL), _f32),
          pltpu.VMEM((BD, NL), _f32),
          pltpu.VMEM((BD, NL), _f32),
          pltpu.VMEM_SHARED((NP, NL), _f32),
          (pltpu.SemaphoreType.DMA,) * 8,
          (pltpu.SemaphoreType.DMA,) * 4,
          (pltpu.SemaphoreType.DMA,) * 4,
      ),
  )
  def k(gidx_h, dst_h, seg_h, y_h, cnt0_h, cnt1_h, oout_h,
        gring, dring, segring, c0ring, c1ring, rb0, rb1, rb2, rb3, o_sh,
        isems, gsems, ssems):
    c = lax.axis_index("c")
    s = lax.axis_index("s")
    nb = jnp.where(c == 0, D_B0, D_B1)
    wb = s * (D_B0 + D_B1) + c * D_B0
    rbs = (rb0, rb1, rb2, rb3)

    zeros16 = jnp.zeros((16,), _f32)

    @pl.loop(0, BD)
    def _(i):
      rb0[i, pl.ds(0, 16)] = zeros16

    base = s * NROW_T
    for off, sz in ((0, BD), (BD, BD), (2 * BD, BD), (3 * BD, BD),
                    (4 * BD, NROW_T - 4 * BD)):
      pltpu.sync_copy(rb0.at[pl.ds(0, sz)], o_sh.at[pl.ds(base + off, sz)])

    def compute_block(r, rb):
      @pl.loop(0, BD, step=16)
      def _(e0):
        ctot = c0ring[r, pl.ds(e0, 16)] + c1ring[r, pl.ds(e0, 16)]
        s_vec = 1.0 / jnp.maximum(ctot, 1.0)
        for l in range(16):
          e = e0 + l
          sbc = _bcast_lane(s_vec, l)
          rb[e, pl.ds(0, 16)] = rb[e, pl.ds(0, 16)] * sbc

    _ring_body(gidx_h, dst_h, seg_h, y_h, cnt0_h, cnt1_h, o_sh,
               gring, dring, segring, c0ring, c1ring, rbs,
               isems, gsems, ssems, nb, wb, compute_block)

    plsc.subcore_barrier()
    for off, sz in ((0, BD), (BD, BD), (2 * BD, BD), (3 * BD, BD),
                    (4 * BD, NROW_T - 4 * BD)):
      pltpu.sync_copy(o_sh.at[pl.ds(base + off, sz)], rb0.at[pl.ds(0, sz)])
      pltpu.sync_copy(rb0.at[pl.ds(0, sz)],
                      oout_h.at[pl.ds(c * NP + base + off, sz)])

  return k(gidx2, dstp, seg, y16, cnt0, cnt1)


def _tc_dense(hpart, root1, bias1, w2cat, root2):
  def body(h_ref, r1_ref, b1_ref, w2_ref, rt2_ref, y_ref, xr_ref):
    x = h_ref[:N, :] + h_ref[NP:NP + N, :] + r1_ref[...] + b1_ref[...]
    x = jnp.maximum(x, 0.0)
    y_ref[...] = jnp.dot(x, w2_ref[...], preferred_element_type=_f32)
    xr_ref[...] = jnp.dot(x, rt2_ref[...], preferred_element_type=_f32)

  return pl.pallas_call(
      body,
      out_shape=(
          jax.ShapeDtypeStruct((N, R * NL), _f32),
          jax.ShapeDtypeStruct((N, NL), _f32),
      ),
  )(hpart, root1, bias1.reshape(1, H), w2cat, root2)


def _tc_final(opart, xr, bias2):
  def body(o_ref, xr_ref, b2_ref, out_ref):
    t = o_ref[:N, :] + o_ref[NP:NP + N, :] + xr_ref[...] + b2_ref[...]
    out_ref[...] = jax.nn.sigmoid(t)

  return pl.pallas_call(
      body,
      out_shape=jax.ShapeDtypeStruct((N, NL), _f32),
  )(opart, xr, bias2.reshape(1, NL))


def kernel(edge_index, edge_type, weight1, root1, bias1, weight2, root2, bias2):
  src = edge_index[0].astype(_i32)
  dst = edge_index[1].astype(_i32)
  rel = edge_type.astype(_i32)

  pad = EP - E
  srcp = jnp.concatenate([src, jnp.zeros((pad,), _i32)]).reshape(EROWS, 128)
  dstp = jnp.concatenate([dst, jnp.full((pad,), N, _i32)]).reshape(EROWS, 128)
  relp = jnp.concatenate([rel, jnp.zeros((pad,), _i32)]).reshape(EROWS, 128)

  w1f = weight1.reshape(NSEG, H)
  w2cat = weight2.transpose(1, 0, 2).reshape(H, R * NL)

  cnt0, cnt1, seg, gidx, gidx2 = _sc_counts(srcp, dstp, relp)
  hpart = _sc_layer1(gidx.reshape(EP // BC, BC), dstp.reshape(EP // BC, BC),
                     seg.reshape(EP // BC, BC), w1f, cnt0, cnt1)
  y2d, xr = _tc_dense(hpart, root1, bias1, w2cat, root2)
  y16 = y2d.reshape(NSEG, NL)
  opart = _sc_layer2(gidx2, dstp, seg, y16, cnt0, cnt1)
  return _tc_final(opart, xr, bias2)


# speedup vs baseline: 1.2040x; 1.0239x over previous
import functools

import jax
import jax.numpy as jnp
from jax import lax
from jax.experimental import pallas as pl
from jax.experimental.pallas import tpu as pltpu
from jax.experimental.pallas import tpu_sc as plsc

N = 10000
R = 8
H = 128
NL = 16
E = 320000

NC = 2
NS = 16
NW = NC * NS
ROWS_W = 80
EPW = ROWS_W * 128
EP = NW * EPW
EROWS = EP // 128
NSEG = N * R
NSEG_P = 80896
SEG_T = NSEG_P // NS
NP = 10112
NROW_T = NP // NS

BC = 64
NBC = EP // BC
C_B0 = 240
C_B1 = 80

BD = 128
NBD = EP // BD
D_B0 = 112
D_B1 = 48

_i32 = jnp.int32
_f32 = jnp.float32


def _mesh():
  return plsc.VectorSubcoreMesh(
      core_axis_name="c", subcore_axis_name="s", num_cores=NC, num_subcores=NS)


def _full16(v):
  return jnp.full((16,), v, dtype=_i32)


def _bcast_lane(vec, l):
  return lax.gather(
      vec, _full16(l).reshape(16, 1),
      lax.GatherDimensionNumbers(
          offset_dims=(), collapsed_slice_dims=(0,), start_index_map=(0,)),
      (1,), mode=lax.GatherScatterMode.PROMISE_IN_BOUNDS)


def _sc_counts(srcp, dstp, relp):
  @functools.partial(
      pl.kernel,
      out_type=(
          jax.ShapeDtypeStruct((NSEG_P,), _f32),
          jax.ShapeDtypeStruct((NSEG_P,), _f32),
          jax.ShapeDtypeStruct((EROWS, 128), _i32),
          jax.ShapeDtypeStruct((EROWS, 128), _i32),
          jax.ShapeDtypeStruct((EROWS, 128), _i32),
      ),
      mesh=_mesh(),
      scratch_types=(
          pltpu.VMEM((ROWS_W, 128), _i32),
          pltpu.VMEM((ROWS_W, 128), _i32),
          pltpu.VMEM((ROWS_W, 128), _i32),
          pltpu.VMEM((ROWS_W, 128), _i32),
          pltpu.VMEM((ROWS_W, 128), _i32),
          pltpu.VMEM((ROWS_W, 128), _i32),
          pltpu.VMEM((128,), _f32),
          pltpu.VMEM((SEG_T,), _f32),
          pltpu.VMEM_SHARED((NSEG_P,), _f32),
          pltpu.SemaphoreType.DMA,
      ),
  )
  def k(src_h, dst_h, rel_h, cnt0_h, cnt1_h, seg_h, gidx_h, gidx2_h,
        sb, db, rb, segb, gb, g2b, ones, zbuf, cnt_sh, sem):
    c = lax.axis_index("c")
    s = lax.axis_index("s")
    wid = s * NC + c
    wb = wid * ROWS_W

    zeros16 = jnp.zeros((16,), _f32)

    @pl.loop(0, SEG_T // 16)
    def _(i):
      zbuf[pl.ds(i * 16, 16)] = zeros16

    pltpu.sync_copy(zbuf, cnt_sh.at[pl.ds(s * SEG_T, SEG_T)])

    for k8 in range(8):
      ones[pl.ds(k8 * 16, 16)] = jnp.ones((16,), _f32)

    pltpu.sync_copy(src_h.at[pl.ds(wb, ROWS_W)], sb)
    pltpu.sync_copy(dst_h.at[pl.ds(wb, ROWS_W)], db)
    pltpu.sync_copy(rel_h.at[pl.ds(wb, ROWS_W)], rb)

    @pl.loop(0, ROWS_W)
    def _(j):
      for k8 in range(8):
        sl = pl.ds(k8 * 16, 16)
        sv = sb[j, sl]
        dv = db[j, sl]
        rv = rb[j, sl]
        segb[j, sl] = dv * R + rv
        gb[j, sl] = rv * N + sv
        g2b[j, sl] = sv * R + rv

    pltpu.sync_copy(segb, seg_h.at[pl.ds(wb, ROWS_W)])
    pltpu.sync_copy(gb, gidx_h.at[pl.ds(wb, ROWS_W)])
    pltpu.sync_copy(g2b, gidx2_h.at[pl.ds(wb, ROWS_W)])

    plsc.subcore_barrier()

    descs = [
        pltpu.async_copy(ones, cnt_sh.at[segb.at[j]], sem, add=True)
        for j in range(ROWS_W)
    ]
    for d in descs:
      d.wait()

    plsc.subcore_barrier()
    pltpu.sync_copy(cnt_sh.at[pl.ds(s * SEG_T, SEG_T)], zbuf)

    @pl.when(c == 0)
    def _():
      pltpu.sync_copy(zbuf, cnt0_h.at[pl.ds(s * SEG_T, SEG_T)])

    @pl.when(c == 1)
    def _():
      pltpu.sync_copy(zbuf, cnt1_h.at[pl.ds(s * SEG_T, SEG_T)])

  return k(srcp, dstp, relp)


def _ring_body(gidx_h, dst_h, seg_h, tab_h, cnt0_h, cnt1_h, acc_sh,
               gring, dring, segring, c0ring, c1ring, rbs,
               isems, gsems, ssems, nb, wb, compute_block):
  def istart(j, slot):
    pltpu.async_copy(gidx_h.at[pl.ds(wb + j, 1)], gring.at[pl.ds(slot, 1)],
                     isems[slot])
    pltpu.async_copy(dst_h.at[pl.ds(wb + j, 1)], dring.at[pl.ds(slot, 1)],
                     isems[slot])
    pltpu.async_copy(seg_h.at[pl.ds(wb + j, 1)], segring.at[pl.ds(slot, 1)],
                     isems[slot])

  def iwait(slot):
    for _ in range(3):
      pltpu.make_async_copy(gidx_h.at[pl.ds(wb, 1)],
                            gring.at[pl.ds(slot, 1)], isems[slot]).wait()

  def gstart(slot, rslot):
    pltpu.async_copy(tab_h.at[gring.at[slot]], rbs[rslot], gsems[rslot])
    pltpu.async_copy(cnt0_h.at[segring.at[slot]], c0ring.at[slot],
                     gsems[rslot])
    pltpu.async_copy(cnt1_h.at[segring.at[slot]], c1ring.at[slot],
                     gsems[rslot])

  def gwait(rslot):
    pltpu.make_async_copy(tab_h.at[gring.at[0]], rbs[rslot],
                          gsems[rslot]).wait()
    for _ in range(2):
      pltpu.make_async_copy(cnt0_h.at[segring.at[0]], c0ring.at[0],
                            gsems[rslot]).wait()

  def sstart(slot, rslot):
    pltpu.async_copy(rbs[rslot], acc_sh.at[dring.at[slot]], ssems[rslot],
                     add=True)

  def swait(rslot):
    pltpu.make_async_copy(rbs[rslot], acc_sh.at[dring.at[0]],
                          ssems[rslot]).wait()

  for p in range(6):
    istart(p, p)
  iwait(0)
  gstart(0, 0)
  iwait(1)
  gstart(1, 1)

  plsc.subcore_barrier()

  @pl.loop(0, nb // 8)
  def _(q):
    for r in range(8):
      j = q * 8 + r
      rs = r % 4

      @pl.when(j >= 2)
      def _():
        swait((rs + 2) % 4)

      @pl.when(j + 2 < nb)
      def _():
        iwait((r + 2) % 8)
        gstart((r + 2) % 8, (rs + 2) % 4)

      @pl.when(j + 6 < nb)
      def _():
        istart(j + 6, (r + 6) % 8)

      gwait(rs)
      compute_block(r, rbs[rs])
      sstart(r, rs)

  swait(2)
  swait(3)


def _sc_layer1(gidx64, dstp64, seg64, w1f, cnt0, cnt1):
  @functools.partial(
      pl.kernel,
      out_type=jax.ShapeDtypeStruct((NC * NP, H), _f32),
      mesh=_mesh(),
      scratch_types=(
          pltpu.VMEM((8, BC), _i32),
          pltpu.VMEM((8, BC), _i32),
          pltpu.VMEM((8, BC), _i32),
          pltpu.VMEM((8, BC), _f32),
          pltpu.VMEM((8, BC), _f32),
          pltpu.VMEM((BC, H), _f32),
          pltpu.VMEM((BC, H), _f32),
          pltpu.VMEM((BC, H), _f32),
          pltpu.VMEM((BC, H), _f32),
          pltpu.VMEM_SHARED((NP, H), _f32),
          (pltpu.SemaphoreType.DMA,) * 8,
          (pltpu.SemaphoreType.DMA,) * 4,
          (pltpu.SemaphoreType.DMA,) * 4,
      ),
  )
  def k(gidx_h, dst_h, seg_h, w1_h, cnt0_h, cnt1_h, hout_h,
        gring, dring, segring, c0ring, c1ring, rb0, rb1, rb2, rb3, h_sh,
        isems, gsems, ssems):
    c = lax.axis_index("c")
    s = lax.axis_index("s")
    nb = jnp.where(c == 0, C_B0, C_B1)
    wb = s * (C_B0 + C_B1) + c * C_B0
    rbs = (rb0, rb1, rb2, rb3)

    zeros16 = jnp.zeros((16,), _f32)

    @pl.loop(0, BC)
    def _(i):
      for k8 in range(8):
        rb0[i, pl.ds(k8 * 16, 16)] = zeros16

    base = s * NROW_T
    off = 0
    for sz in [BC] * 9 + [NROW_T - 9 * BC]:
      pltpu.sync_copy(rb0.at[pl.ds(0, sz)], h_sh.at[pl.ds(base + off, sz)])
      off += sz

    def compute_block(r, rb):
      @pl.loop(0, BC, step=16)
      def _(e0):
        ctot = c0ring[r, pl.ds(e0, 16)] + c1ring[r, pl.ds(e0, 16)]
        s_vec = 1.0 / jnp.maximum(ctot, 1.0)
        for l in range(16):
          e = e0 + l
          sbc = _bcast_lane(s_vec, l)
          for k8 in range(8):
            sl = pl.ds(k8 * 16, 16)
            rb[e, sl] = rb[e, sl] * sbc

    _ring_body(gidx_h, dst_h, seg_h, w1_h, cnt0_h, cnt1_h, h_sh,
               gring, dring, segring, c0ring, c1ring, rbs,
               isems, gsems, ssems, nb, wb, compute_block)

    plsc.subcore_barrier()
    off = 0
    for sz in [BC] * 9 + [NROW_T - 9 * BC]:
      pltpu.sync_copy(h_sh.at[pl.ds(base + off, sz)], rb0.at[pl.ds(0, sz)])
      pltpu.sync_copy(rb0.at[pl.ds(0, sz)],
                      hout_h.at[pl.ds(c * NP + base + off, sz)])
      off += sz

  return k(gidx64, dstp64, seg64, w1f, cnt0, cnt1)


def _sc_layer2(gidx2, dstp, seg, y16, cnt0, cnt1):
  @functools.partial(
      pl.kernel,
      out_type=jax.ShapeDtypeStruct((NC * NP, NL), _f32),
      mesh=_mesh(),
      compiler_params=pltpu.CompilerParams(use_tc_tiling_on_sc=False),
      scratch_types=(
          pltpu.VMEM((8, BD), _i32),
          pltpu.VMEM((8, BD), _i32),
          pltpu.VMEM((8, BD), _i32),
          pltpu.VMEM((8, BD), _f32),
          pltpu.VMEM((8, BD), _f32),
          pltpu.VMEM((BD, NL), _f32),
          pltpu.VMEM((BD, NL), _f32),
          pltpu.VMEM((BD, NL), _f32),
          pltpu.VMEM((BD, NL), _f32),
          pltpu.VMEM_SHARED((NP, NL), _f32),
          (pltpu.SemaphoreType.DMA,) * 8,
          (pltpu.SemaphoreType.DMA,) * 4,
          (pltpu.SemaphoreType.DMA,) * 4,
      ),
  )
  def k(gidx_h, dst_h, seg_h, y_h, cnt0_h, cnt1_h, oout_h,
        gring, dring, segring, c0ring, c1ring, rb0, rb1, rb2, rb3, o_sh,
        isems, gsems, ssems):
    c = lax.axis_index("c")
    s = lax.axis_index("s")
    nb = jnp.where(c == 0, D_B0, D_B1)
    wb = s * (D_B0 + D_B1) + c * D_B0
    rbs = (rb0, rb1, rb2, rb3)

    zeros16 = jnp.zeros((16,), _f32)

    @pl.loop(0, BD)
    def _(i):
      rb0[i, pl.ds(0, 16)] = zeros16

    base = s * NROW_T
    for off, sz in ((0, BD), (BD, BD), (2 * BD, BD), (3 * BD, BD),
                    (4 * BD, NROW_T - 4 * BD)):
      pltpu.sync_copy(rb0.at[pl.ds(0, sz)], o_sh.at[pl.ds(base + off, sz)])

    def compute_block(r, rb):
      @pl.loop(0, BD, step=16)
      def _(e0):
        ctot = c0ring[r, pl.ds(e0, 16)] + c1ring[r, pl.ds(e0, 16)]
        s_vec = 1.0 / jnp.maximum(ctot, 1.0)
        for l in range(16):
          e = e0 + l
          sbc = _bcast_lane(s_vec, l)
          rb[e, pl.ds(0, 16)] = rb[e, pl.ds(0, 16)] * sbc

    _ring_body(gidx_h, dst_h, seg_h, y_h, cnt0_h, cnt1_h, o_sh,
               gring, dring, segring, c0ring, c1ring, rbs,
               isems, gsems, ssems, nb, wb, compute_block)

    plsc.subcore_barrier()
    for off, sz in ((0, BD), (BD, BD), (2 * BD, BD), (3 * BD, BD),
                    (4 * BD, NROW_T - 4 * BD)):
      pltpu.sync_copy(o_sh.at[pl.ds(base + off, sz)], rb0.at[pl.ds(0, sz)])
      pltpu.sync_copy(rb0.at[pl.ds(0, sz)],
                      oout_h.at[pl.ds(c * NP + base + off, sz)])

  return k(gidx2, dstp, seg, y16, cnt0, cnt1)


def _tc_dense(hpart, root1, bias1, w2cat, root2):
  def body(h_ref, r1_ref, b1_ref, w2_ref, rt2_ref, y_ref, xr_ref):
    x = h_ref[:N, :] + h_ref[NP:NP + N, :] + r1_ref[...] + b1_ref[...]
    x = jnp.maximum(x, 0.0)
    y_ref[...] = jnp.dot(x, w2_ref[...], preferred_element_type=_f32)
    xr_ref[...] = jnp.dot(x, rt2_ref[...], preferred_element_type=_f32)

  return pl.pallas_call(
      body,
      out_shape=(
          jax.ShapeDtypeStruct((N, R * NL), _f32),
          jax.ShapeDtypeStruct((N, NL), _f32),
      ),
  )(hpart, root1, bias1.reshape(1, H), w2cat, root2)


def _tc_final(opart, xr, bias2):
  def body(o_ref, xr_ref, b2_ref, out_ref):
    t = o_ref[:N, :] + o_ref[NP:NP + N, :] + xr_ref[...] + b2_ref[...]
    out_ref[...] = jax.nn.sigmoid(t)

  return pl.pallas_call(
      body,
      out_shape=jax.ShapeDtypeStruct((N, NL), _f32),
  )(opart, xr, bias2.reshape(1, NL))


def kernel(edge_index, edge_type, weight1, root1, bias1, weight2, root2, bias2):
  src = edge_index[0].astype(_i32)
  dst = edge_index[1].astype(_i32)
  rel = edge_type.astype(_i32)

  pad = EP - E
  trash = N + jnp.arange(pad, dtype=_i32) % (NP - N)
  srcp = jnp.concatenate([src, jnp.zeros((pad,), _i32)]).reshape(EROWS, 128)
  dstp = jnp.concatenate([dst, trash]).reshape(EROWS, 128)
  relp = jnp.concatenate([rel, jnp.zeros((pad,), _i32)]).reshape(EROWS, 128)

  w1f = weight1.reshape(NSEG, H)
  w2cat = weight2.transpose(1, 0, 2).reshape(H, R * NL)

  cnt0, cnt1, seg, gidx, gidx2 = _sc_counts(srcp, dstp, relp)
  hpart = _sc_layer1(gidx.reshape(EP // BC, BC), dstp.reshape(EP // BC, BC),
                     seg.reshape(EP // BC, BC), w1f, cnt0, cnt1)
  y2d, xr = _tc_dense(hpart, root1, bias1, w2cat, root2)
  y16 = y2d.reshape(NSEG, NL)
  opart = _sc_layer2(gidx2, dstp, seg, y16, cnt0, cnt1)
  return _tc_final(opart, xr, bias2)


# speedup vs baseline: 1.2041x; 1.0001x over previous
import functools

import jax
import jax.numpy as jnp
from jax import lax
from jax.experimental import pallas as pl
from jax.experimental.pallas import tpu as pltpu
from jax.experimental.pallas import tpu_sc as plsc

N = 10000
R = 8
H = 128
NL = 16
E = 320000

NC = 2
NS = 16
NW = NC * NS
ROWS_W = 80
EPW = ROWS_W * 128
EP = NW * EPW
EROWS = EP // 128
NSEG = N * R
NSEG_P = 80896
SEG_T = NSEG_P // NS
NP = 10112
NROW_T = NP // NS

BC = 80
NBC = EP // BC
C_B0 = 192
C_B1 = 64

BD = 128
NBD = EP // BD
D_B0 = 112
D_B1 = 48

_i32 = jnp.int32
_f32 = jnp.float32


def _mesh():
  return plsc.VectorSubcoreMesh(
      core_axis_name="c", subcore_axis_name="s", num_cores=NC, num_subcores=NS)


def _full16(v):
  return jnp.full((16,), v, dtype=_i32)


def _bcast_lane(vec, l):
  return lax.gather(
      vec, _full16(l).reshape(16, 1),
      lax.GatherDimensionNumbers(
          offset_dims=(), collapsed_slice_dims=(0,), start_index_map=(0,)),
      (1,), mode=lax.GatherScatterMode.PROMISE_IN_BOUNDS)


def _sc_counts(srcp, dstp, relp):
  @functools.partial(
      pl.kernel,
      out_type=(
          jax.ShapeDtypeStruct((NSEG_P,), _f32),
          jax.ShapeDtypeStruct((NSEG_P,), _f32),
          jax.ShapeDtypeStruct((EROWS, 128), _i32),
          jax.ShapeDtypeStruct((EROWS, 128), _i32),
          jax.ShapeDtypeStruct((EROWS, 128), _i32),
      ),
      mesh=_mesh(),
      scratch_types=(
          pltpu.VMEM((ROWS_W, 128), _i32),
          pltpu.VMEM((ROWS_W, 128), _i32),
          pltpu.VMEM((ROWS_W, 128), _i32),
          pltpu.VMEM((ROWS_W, 128), _i32),
          pltpu.VMEM((ROWS_W, 128), _i32),
          pltpu.VMEM((ROWS_W, 128), _i32),
          pltpu.VMEM((128,), _f32),
          pltpu.VMEM((SEG_T,), _f32),
          pltpu.VMEM_SHARED((NSEG_P,), _f32),
          pltpu.SemaphoreType.DMA,
      ),
  )
  def k(src_h, dst_h, rel_h, cnt0_h, cnt1_h, seg_h, gidx_h, gidx2_h,
        sb, db, rb, segb, gb, g2b, ones, zbuf, cnt_sh, sem):
    c = lax.axis_index("c")
    s = lax.axis_index("s")
    wid = s * NC + c
    wb = wid * ROWS_W

    zeros16 = jnp.zeros((16,), _f32)

    @pl.loop(0, SEG_T // 16)
    def _(i):
      zbuf[pl.ds(i * 16, 16)] = zeros16

    pltpu.sync_copy(zbuf, cnt_sh.at[pl.ds(s * SEG_T, SEG_T)])

    for k8 in range(8):
      ones[pl.ds(k8 * 16, 16)] = jnp.ones((16,), _f32)

    pltpu.sync_copy(src_h.at[pl.ds(wb, ROWS_W)], sb)
    pltpu.sync_copy(dst_h.at[pl.ds(wb, ROWS_W)], db)
    pltpu.sync_copy(rel_h.at[pl.ds(wb, ROWS_W)], rb)

    @pl.loop(0, ROWS_W)
    def _(j):
      for k8 in range(8):
        sl = pl.ds(k8 * 16, 16)
        sv = sb[j, sl]
        dv = db[j, sl]
        rv = rb[j, sl]
        segb[j, sl] = dv * R + rv
        gb[j, sl] = rv * N + sv
        g2b[j, sl] = sv * R + rv

    pltpu.sync_copy(segb, seg_h.at[pl.ds(wb, ROWS_W)])
    pltpu.sync_copy(gb, gidx_h.at[pl.ds(wb, ROWS_W)])
    pltpu.sync_copy(g2b, gidx2_h.at[pl.ds(wb, ROWS_W)])

    plsc.subcore_barrier()

    descs = [
        pltpu.async_copy(ones, cnt_sh.at[segb.at[j]], sem, add=True)
        for j in range(ROWS_W)
    ]
    for d in descs:
      d.wait()

    plsc.subcore_barrier()
    pltpu.sync_copy(cnt_sh.at[pl.ds(s * SEG_T, SEG_T)], zbuf)

    @pl.when(c == 0)
    def _():
      pltpu.sync_copy(zbuf, cnt0_h.at[pl.ds(s * SEG_T, SEG_T)])

    @pl.when(c == 1)
    def _():
      pltpu.sync_copy(zbuf, cnt1_h.at[pl.ds(s * SEG_T, SEG_T)])

  return k(srcp, dstp, relp)


def _ring_body(gidx_h, dst_h, seg_h, tab_h, cnt0_h, cnt1_h, acc_sh,
               gring, dring, segring, c0ring, c1ring, rbs,
               isems, gsems, ssems, nb, wb, compute_block):
  def istart(j, slot):
    pltpu.async_copy(gidx_h.at[pl.ds(wb + j, 1)], gring.at[pl.ds(slot, 1)],
                     isems[slot])
    pltpu.async_copy(dst_h.at[pl.ds(wb + j, 1)], dring.at[pl.ds(slot, 1)],
                     isems[slot])
    pltpu.async_copy(seg_h.at[pl.ds(wb + j, 1)], segring.at[pl.ds(slot, 1)],
                     isems[slot])

  def iwait(slot):
    for _ in range(3):
      pltpu.make_async_copy(gidx_h.at[pl.ds(wb, 1)],
                            gring.at[pl.ds(slot, 1)], isems[slot]).wait()

  def gstart(slot, rslot):
    pltpu.async_copy(tab_h.at[gring.at[slot]], rbs[rslot], gsems[rslot])
    pltpu.async_copy(cnt0_h.at[segring.at[slot]], c0ring.at[slot],
                     gsems[rslot])
    pltpu.async_copy(cnt1_h.at[segring.at[slot]], c1ring.at[slot],
                     gsems[rslot])

  def gwait(rslot):
    pltpu.make_async_copy(tab_h.at[gring.at[0]], rbs[rslot],
                          gsems[rslot]).wait()
    for _ in range(2):
      pltpu.make_async_copy(cnt0_h.at[segring.at[0]], c0ring.at[0],
                            gsems[rslot]).wait()

  def sstart(slot, rslot):
    pltpu.async_copy(rbs[rslot], acc_sh.at[dring.at[slot]], ssems[rslot],
                     add=True)

  def swait(rslot):
    pltpu.make_async_copy(rbs[rslot], acc_sh.at[dring.at[0]],
                          ssems[rslot]).wait()

  for p in range(6):
    istart(p, p)
  iwait(0)
  gstart(0, 0)
  iwait(1)
  gstart(1, 1)

  plsc.subcore_barrier()

  @pl.loop(0, nb // 8)
  def _(q):
    for r in range(8):
      j = q * 8 + r
      rs = r % 4

      @pl.when(j >= 2)
      def _():
        swait((rs + 2) % 4)

      @pl.when(j + 2 < nb)
      def _():
        iwait((r + 2) % 8)
        gstart((r + 2) % 8, (rs + 2) % 4)

      @pl.when(j + 6 < nb)
      def _():
        istart(j + 6, (r + 6) % 8)

      gwait(rs)
      compute_block(r, rbs[rs])
      sstart(r, rs)

  swait(2)
  swait(3)


def _sc_layer1(gidx64, dstp64, seg64, w1f, cnt0, cnt1):
  @functools.partial(
      pl.kernel,
      out_type=jax.ShapeDtypeStruct((NC * NP, H), _f32),
      mesh=_mesh(),
      scratch_types=(
          pltpu.VMEM((8, BC), _i32),
          pltpu.VMEM((8, BC), _i32),
          pltpu.VMEM((8, BC), _i32),
          pltpu.VMEM((8, BC), _f32),
          pltpu.VMEM((8, BC), _f32),
          pltpu.VMEM((BC, H), _f32),
          pltpu.VMEM((BC, H), _f32),
          pltpu.VMEM((BC, H), _f32),
          pltpu.VMEM((BC, H), _f32),
          pltpu.VMEM_SHARED((NP, H), _f32),
          (pltpu.SemaphoreType.DMA,) * 8,
          (pltpu.SemaphoreType.DMA,) * 4,
          (pltpu.SemaphoreType.DMA,) * 4,
      ),
  )
  def k(gidx_h, dst_h, seg_h, w1_h, cnt0_h, cnt1_h, hout_h,
        gring, dring, segring, c0ring, c1ring, rb0, rb1, rb2, rb3, h_sh,
        isems, gsems, ssems):
    c = lax.axis_index("c")
    s = lax.axis_index("s")
    nb = jnp.where(c == 0, C_B0, C_B1)
    wb = s * (C_B0 + C_B1) + c * C_B0
    rbs = (rb0, rb1, rb2, rb3)

    zeros16 = jnp.zeros((16,), _f32)

    @pl.loop(0, BC)
    def _(i):
      for k8 in range(8):
        rb0[i, pl.ds(k8 * 16, 16)] = zeros16

    base = s * NROW_T
    off = 0
    zcs = [BC] * (NROW_T // BC) + ([NROW_T % BC] if NROW_T % BC else [])
    for sz in zcs:
      pltpu.sync_copy(rb0.at[pl.ds(0, sz)], h_sh.at[pl.ds(base + off, sz)])
      off += sz

    def compute_block(r, rb):
      @pl.loop(0, BC, step=16)
      def _(e0):
        ctot = c0ring[r, pl.ds(e0, 16)] + c1ring[r, pl.ds(e0, 16)]
        s_vec = 1.0 / jnp.maximum(ctot, 1.0)
        for l in range(16):
          e = e0 + l
          sbc = _bcast_lane(s_vec, l)
          for k8 in range(8):
            sl = pl.ds(k8 * 16, 16)
            rb[e, sl] = rb[e, sl] * sbc

    _ring_body(gidx_h, dst_h, seg_h, w1_h, cnt0_h, cnt1_h, h_sh,
               gring, dring, segring, c0ring, c1ring, rbs,
               isems, gsems, ssems, nb, wb, compute_block)

    plsc.subcore_barrier()
    off = 0
    for sz in zcs:
      pltpu.sync_copy(h_sh.at[pl.ds(base + off, sz)], rb0.at[pl.ds(0, sz)])
      pltpu.sync_copy(rb0.at[pl.ds(0, sz)],
                      hout_h.at[pl.ds(c * NP + base + off, sz)])
      off += sz

  return k(gidx64, dstp64, seg64, w1f, cnt0, cnt1)


def _sc_layer2(gidx2, dstp, seg, y16, cnt0, cnt1):
  @functools.partial(
      pl.kernel,
      out_type=jax.ShapeDtypeStruct((NC * NP, NL), _f32),
      mesh=_mesh(),
      compiler_params=pltpu.CompilerParams(use_tc_tiling_on_sc=False),
      scratch_types=(
          pltpu.VMEM((8, BD), _i32),
          pltpu.VMEM((8, BD), _i32),
          pltpu.VMEM((8, BD), _i32),
          pltpu.VMEM((8, BD), _f32),
          pltpu.VMEM((8, BD), _f32),
          pltpu.VMEM((BD, NL), _f32),
          pltpu.VMEM((BD, NL), _f32),
          pltpu.VMEM((BD, NL), _f32),
          pltpu.VMEM((BD, NL), _f32),
          pltpu.VMEM_SHARED((NP, NL), _f32),
          (pltpu.SemaphoreType.DMA,) * 8,
          (pltpu.SemaphoreType.DMA,) * 4,
          (pltpu.SemaphoreType.DMA,) * 4,
      ),
  )
  def k(gidx_h, dst_h, seg_h, y_h, cnt0_h, cnt1_h, oout_h,
        gring, dring, segring, c0ring, c1ring, rb0, rb1, rb2, rb3, o_sh,
        isems, gsems, ssems):
    c = lax.axis_index("c")
    s = lax.axis_index("s")
    nb = jnp.where(c == 0, D_B0, D_B1)
    wb = s * (D_B0 + D_B1) + c * D_B0
    rbs = (rb0, rb1, rb2, rb3)

    zeros16 = jnp.zeros((16,), _f32)

    @pl.loop(0, BD)
    def _(i):
      rb0[i, pl.ds(0, 16)] = zeros16

    base = s * NROW_T
    for off, sz in ((0, BD), (BD, BD), (2 * BD, BD), (3 * BD, BD),
                    (4 * BD, NROW_T - 4 * BD)):
      pltpu.sync_copy(rb0.at[pl.ds(0, sz)], o_sh.at[pl.ds(base + off, sz)])

    def compute_block(r, rb):
      @pl.loop(0, BD, step=16)
      def _(e0):
        ctot = c0ring[r, pl.ds(e0, 16)] + c1ring[r, pl.ds(e0, 16)]
        s_vec = 1.0 / jnp.maximum(ctot, 1.0)
        for l in range(16):
          e = e0 + l
          sbc = _bcast_lane(s_vec, l)
          rb[e, pl.ds(0, 16)] = rb[e, pl.ds(0, 16)] * sbc

    _ring_body(gidx_h, dst_h, seg_h, y_h, cnt0_h, cnt1_h, o_sh,
               gring, dring, segring, c0ring, c1ring, rbs,
               isems, gsems, ssems, nb, wb, compute_block)

    plsc.subcore_barrier()
    for off, sz in ((0, BD), (BD, BD), (2 * BD, BD), (3 * BD, BD),
                    (4 * BD, NROW_T - 4 * BD)):
      pltpu.sync_copy(o_sh.at[pl.ds(base + off, sz)], rb0.at[pl.ds(0, sz)])
      pltpu.sync_copy(rb0.at[pl.ds(0, sz)],
                      oout_h.at[pl.ds(c * NP + base + off, sz)])

  return k(gidx2, dstp, seg, y16, cnt0, cnt1)


def _tc_dense(hpart, root1, bias1, w2cat, root2):
  def body(h_ref, r1_ref, b1_ref, w2_ref, rt2_ref, y_ref, xr_ref):
    x = h_ref[:N, :] + h_ref[NP:NP + N, :] + r1_ref[...] + b1_ref[...]
    x = jnp.maximum(x, 0.0)
    y_ref[...] = jnp.dot(x, w2_ref[...], preferred_element_type=_f32)
    xr_ref[...] = jnp.dot(x, rt2_ref[...], preferred_element_type=_f32)

  return pl.pallas_call(
      body,
      out_shape=(
          jax.ShapeDtypeStruct((N, R * NL), _f32),
          jax.ShapeDtypeStruct((N, NL), _f32),
      ),
  )(hpart, root1, bias1.reshape(1, H), w2cat, root2)


def _tc_final(opart, xr, bias2):
  def body(o_ref, xr_ref, b2_ref, out_ref):
    t = o_ref[:N, :] + o_ref[NP:NP + N, :] + xr_ref[...] + b2_ref[...]
    out_ref[...] = jax.nn.sigmoid(t)

  return pl.pallas_call(
      body,
      out_shape=jax.ShapeDtypeStruct((N, NL), _f32),
  )(opart, xr, bias2.reshape(1, NL))


def kernel(edge_index, edge_type, weight1, root1, bias1, weight2, root2, bias2):
  src = edge_index[0].astype(_i32)
  dst = edge_index[1].astype(_i32)
  rel = edge_type.astype(_i32)

  pad = EP - E
  trash = N + jnp.arange(pad, dtype=_i32) % (NP - N)
  srcp = jnp.concatenate([src, jnp.zeros((pad,), _i32)]).reshape(EROWS, 128)
  dstp = jnp.concatenate([dst, trash]).reshape(EROWS, 128)
  relp = jnp.concatenate([rel, jnp.zeros((pad,), _i32)]).reshape(EROWS, 128)

  w1f = weight1.reshape(NSEG, H)
  w2cat = weight2.transpose(1, 0, 2).reshape(H, R * NL)

  cnt0, cnt1, seg, gidx, gidx2 = _sc_counts(srcp, dstp, relp)
  hpart = _sc_layer1(gidx.reshape(EP // BC, BC), dstp.reshape(EP // BC, BC),
                     seg.reshape(EP // BC, BC), w1f, cnt0, cnt1)
  y2d, xr = _tc_dense(hpart, root1, bias1, w2cat, root2)
  y16 = y2d.reshape(NSEG, NL)
  opart = _sc_layer2(gidx2, dstp, seg, y16, cnt0, cnt1)
  return _tc_final(opart, xr, bias2)
